# baseline (speedup 1.0000x reference)
import functools

import jax
import jax.numpy as jnp
from jax import lax
from jax.experimental import pallas as pl
from jax.experimental.pallas import tpu as pltpu
from jax.experimental.pallas import tpu_sc as plsc

N = 10000
E = 320000
IN_DIM = 128
H = 64
OUT_DIM = 40
TAU = 1.0
MAX_DEG = 4

NP = 10240
NW = 32
CHUNK = 128
NCH = 80
EPW = NCH * CHUNK
EPAD = NW * EPW
DEPTH = 5
RPT = NP // 16
DW = 16

_HI = lax.Precision.HIGHEST

_sc_mesh = plsc.VectorSubcoreMesh(core_axis_name="c", subcore_axis_name="s",
                                  num_cores=2, num_subcores=16)


@functools.partial(
    pl.kernel,
    out_type=jax.ShapeDtypeStruct((2, NP, H), jnp.float32),
    mesh=_sc_mesh,
    scratch_types=[
        pltpu.VMEM((NCH, CHUNK), jnp.int32),
        pltpu.VMEM((NCH, CHUNK), jnp.int32),
        [pltpu.VMEM((CHUNK, H), jnp.float32)] * DEPTH,
        pltpu.VMEM_SHARED((NP, H), jnp.float32),
        [pltpu.SemaphoreType.DMA] * DEPTH,
        [pltpu.SemaphoreType.DMA] * DEPTH,
    ],
    compiler_params=pltpu.CompilerParams(use_tc_tiling_on_sc=False),
)
def _sc_aggregate(z_hbm, src_hbm, dst_hbm, zeros_hbm, out_hbm,
                  src_v, dst_v, bufs, agg, gsems, ssems):
    c = lax.axis_index("c")
    s = lax.axis_index("s")
    wid = c * 16 + s
    st0 = pltpu.async_copy(src_hbm.at[wid], src_v, gsems[0])
    st1 = pltpu.async_copy(dst_hbm.at[wid], dst_v, gsems[1])
    st2 = pltpu.async_copy(zeros_hbm, agg.at[pl.ds(s * RPT, RPT)], gsems[2])
    st0.wait()
    st1.wait()
    st2.wait()
    plsc.subcore_barrier()

    for b in range(DEPTH):
        pltpu.async_copy(z_hbm.at[src_v.at[b]], bufs[b], gsems[b])

    def body(g, carry):
        j0 = g * DEPTH
        for b in range(DEPTH):
            j = j0 + b
            pltpu.make_async_copy(z_hbm.at[src_v.at[j]], bufs[b],
                                  gsems[b]).wait()
            pltpu.async_copy(bufs[b], agg.at[dst_v.at[j]], ssems[b],
                             add=True)
        for b in range(DEPTH):
            j = j0 + b
            pltpu.make_async_copy(bufs[b], agg.at[dst_v.at[j]],
                                  ssems[b]).wait()

            @pl.when(j + DEPTH < NCH)
            def _():
                pltpu.async_copy(z_hbm.at[src_v.at[j + DEPTH]], bufs[b],
                                 gsems[b])
        return carry

    lax.fori_loop(0, NCH // DEPTH, body, 0)
    plsc.subcore_barrier()
    pltpu.sync_copy(agg.at[pl.ds(s * RPT, RPT)],
                    out_hbm.at[c, pl.ds(s * RPT, RPT)])


@functools.partial(
    pl.kernel,
    out_type=jax.ShapeDtypeStruct((2, NP, DW), jnp.float32),
    mesh=_sc_mesh,
    scratch_types=[
        pltpu.VMEM((NCH, CHUNK), jnp.int32),
        pltpu.VMEM((CHUNK, DW), jnp.float32),
        pltpu.VMEM_SHARED((NP, DW), jnp.float32),
        [pltpu.SemaphoreType.DMA] * 2,
    ],
    compiler_params=pltpu.CompilerParams(use_tc_tiling_on_sc=False),
)
def _sc_degree(dst_hbm, ones_hbm, zeros_hbm, out_hbm, dst_v, ones_v, deg,
               sems):
    c = lax.axis_index("c")
    s = lax.axis_index("s")
    wid = c * 16 + s
    st0 = pltpu.async_copy(dst_hbm.at[wid], dst_v, sems[0])
    st1 = pltpu.async_copy(ones_hbm, ones_v, sems[1])
    pltpu.sync_copy(zeros_hbm, deg.at[pl.ds(s * RPT, RPT)])
    st0.wait()
    st1.wait()
    plsc.subcore_barrier()

    def body(j, carry):
        pltpu.sync_copy(ones_v, deg.at[dst_v.at[j]], add=True)
        return carry

    lax.fori_loop(0, NCH, body, 0)
    plsc.subcore_barrier()
    pltpu.sync_copy(deg.at[pl.ds(s * RPT, RPT)],
                    out_hbm.at[c, pl.ds(s * RPT, RPT)])


_GRID = 10
_BN = NP // _GRID


def _row_spec(d):
    return pl.BlockSpec((_BN, d), lambda i: (i, 0))


def _w_spec(a, b):
    return pl.BlockSpec((a, b), lambda i: (0, 0))


_AGG_SPEC = pl.BlockSpec((2, _BN, H), lambda i: (0, i, 0))
_DEG_SPEC = pl.BlockSpec((2, _BN, DW), lambda i: (0, i, 0))


def _rot_fwd(c, s, h):
    h0 = h[:, : H // 2]
    h1 = h[:, H // 2:]
    return jnp.concatenate([c * h0 + s * h1, c * h1 - s * h0], axis=1)


def _rot_bwd(c, s, z):
    z0 = z[:, : H // 2]
    z1 = z[:, H // 2:]
    return jnp.concatenate([c * z0 - s * z1, s * z0 + c * z1], axis=1)


def _theta_head(h, eW, eb, dW, db):
    t = jnp.dot(h, eW, preferred_element_type=jnp.float32, precision=_HI) + eb
    return jnp.dot(t, dW, preferred_element_type=jnp.float32, precision=_HI) + db


def _tc1_body(x_r, degp_r, encW_r, encb_r, eW_r, eb_r, dW_r, db_r,
              h_r, dinv_r, c_r, s_r, y_r, zin_r):
    deg2 = degp_r[0] + degp_r[1]
    dinv2 = jnp.where(deg2 > 0.0, 1.0 / jnp.sqrt(deg2), 0.0)
    dinv = dinv2[:, 0:1]
    h = jnp.dot(x_r[...], encW_r[...], preferred_element_type=jnp.float32,
                precision=_HI) + encb_r[...]
    theta = _theta_head(h, eW_r[...], eb_r[...], dW_r[...], db_r[...])
    c = jnp.cos(theta)
    s = jnp.sin(theta)
    y = _rot_fwd(c, s, h)
    h_r[...] = h
    dinv_r[...] = dinv
    c_r[...] = c
    s_r[...] = s
    y_r[...] = y
    zin_r[...] = dinv * y


_tc1 = pl.pallas_call(
    _tc1_body,
    grid=(_GRID,),
    in_specs=[_row_spec(IN_DIM), _DEG_SPEC, _w_spec(IN_DIM, H), _w_spec(1, H),
              _w_spec(H, H), _w_spec(1, H), _w_spec(H, 1), _w_spec(1, 1)],
    out_specs=[_row_spec(H), _row_spec(1), _row_spec(1), _row_spec(1),
               _row_spec(H), _row_spec(H)],
    out_shape=[jax.ShapeDtypeStruct((NP, H), jnp.float32),
               jax.ShapeDtypeStruct((NP, 1), jnp.float32),
               jax.ShapeDtypeStruct((NP, 1), jnp.float32),
               jax.ShapeDtypeStruct((NP, 1), jnp.float32),
               jax.ShapeDtypeStruct((NP, H), jnp.float32),
               jax.ShapeDtypeStruct((NP, H), jnp.float32)],
)


def _make_tc_step(k):
    def body(term_r, out_r, aggp_r, dinv_r, term2_r, out2_r, zin2_r):
        dinv = dinv_r[...]
        ahat = dinv * (aggp_r[0] + aggp_r[1])
        term2 = (-TAU / k) * (term_r[...] - ahat)
        term2_r[...] = term2
        out2_r[...] = out_r[...] + term2
        zin2_r[...] = dinv * term2

    return pl.pallas_call(
        body,
        grid=(_GRID,),
        in_specs=[_row_spec(H), _row_spec(H), _AGG_SPEC, _row_spec(1)],
        out_specs=[_row_spec(H), _row_spec(H), _row_spec(H)],
        out_shape=[jax.ShapeDtypeStruct((NP, H), jnp.float32)] * 3,
    )


_tc_steps = {k: _make_tc_step(k) for k in (1, 2, 3)}


def _finish_layer(term, out, aggp, dinv, c, s, h, convW, convb):
    ahat = dinv * (aggp[0] + aggp[1])
    z = out + (-TAU / MAX_DEG) * (term - ahat)
    w = _rot_bwd(c, s, z)
    hn = jnp.dot(w, convW, preferred_element_type=jnp.float32,
                 precision=_HI) + convb + h
    return jnp.maximum(hn, 0.0)


def _tc_mid_body(term_r, out_r, aggp_r, dinv_r, c_r, s_r, h_r,
                 convW_r, convb_r, eW_r, eb_r, dW_r, db_r,
                 hn_r, c1_r, s1_r, y1_r, zin1_r):
    dinv = dinv_r[...]
    hn = _finish_layer(term_r[...], out_r[...], aggp_r, dinv,
                       c_r[...], s_r[...], h_r[...], convW_r[...], convb_r[...])
    theta = _theta_head(hn, eW_r[...], eb_r[...], dW_r[...], db_r[...])
    c1 = jnp.cos(theta)
    s1 = jnp.sin(theta)
    y1 = _rot_fwd(c1, s1, hn)
    hn_r[...] = hn
    c1_r[...] = c1
    s1_r[...] = s1
    y1_r[...] = y1
    zin1_r[...] = dinv * y1


_tc_mid = pl.pallas_call(
    _tc_mid_body,
    grid=(_GRID,),
    in_specs=[_row_spec(H), _row_spec(H), _AGG_SPEC, _row_spec(1),
              _row_spec(1), _row_spec(1), _row_spec(H),
              _w_spec(H, H), _w_spec(1, H),
              _w_spec(H, H), _w_spec(1, H), _w_spec(H, 1), _w_spec(1, 1)],
    out_specs=[_row_spec(H), _row_spec(1), _row_spec(1), _row_spec(H),
               _row_spec(H)],
    out_shape=[jax.ShapeDtypeStruct((NP, H), jnp.float32),
               jax.ShapeDtypeStruct((NP, 1), jnp.float32),
               jax.ShapeDtypeStruct((NP, 1), jnp.float32),
               jax.ShapeDtypeStruct((NP, H), jnp.float32),
               jax.ShapeDtypeStruct((NP, H), jnp.float32)],
)


def _tc_final_body(term_r, out_r, aggp_r, dinv_r, c_r, s_r, h_r,
                   convW_r, convb_r, decW_r, decb_r, o_r):
    hn = _finish_layer(term_r[...], out_r[...], aggp_r, dinv_r[...],
                       c_r[...], s_r[...], h_r[...], convW_r[...], convb_r[...])
    o_r[...] = jnp.dot(hn, decW_r[...], preferred_element_type=jnp.float32,
                       precision=_HI) + decb_r[...]


_tc_final = pl.pallas_call(
    _tc_final_body,
    grid=(_GRID,),
    in_specs=[_row_spec(H), _row_spec(H), _AGG_SPEC, _row_spec(1),
              _row_spec(1), _row_spec(1), _row_spec(H),
              _w_spec(H, H), _w_spec(1, H),
              _w_spec(H, OUT_DIM), _w_spec(1, OUT_DIM)],
    out_specs=[_row_spec(OUT_DIM)],
    out_shape=[jax.ShapeDtypeStruct((NP, OUT_DIM), jnp.float32)],
)


def kernel(x, edge_index, enc_W, enc_b, ne0_eW, ne0_eb, ne0_dW, ne0_db,
           conv0_W, conv0_b, ne1_eW, ne1_eb, ne1_dW, ne1_db,
           conv1_W, conv1_b, dec_W, dec_b):
    f32 = jnp.float32
    pad_idx = (N + (jnp.arange(EPAD - E, dtype=jnp.int32) % (NP - N)))
    src3 = jnp.concatenate([edge_index[0], pad_idx]).reshape(NW, NCH, CHUNK)
    dst3 = jnp.concatenate([edge_index[1], pad_idx]).reshape(NW, NCH, CHUNK)
    x = jnp.pad(x, ((0, NP - N), (0, 0)))
    ones_tab = jnp.ones((NP, H), f32)
    zeros_tile = jnp.zeros((RPT, H), f32)
    ones_rows = jnp.ones((CHUNK, DW), f32)
    zeros_deg = jnp.zeros((RPT, DW), f32)

    encb = enc_b.reshape(1, H)
    e0b = ne0_eb.reshape(1, H)
    d0b = ne0_db.reshape(1, 1)
    c0b = conv0_b.reshape(1, H)
    e1b = ne1_eb.reshape(1, H)
    d1b = ne1_db.reshape(1, 1)
    c1b = conv1_b.reshape(1, H)
    decb = dec_b.reshape(1, OUT_DIM)

    degp = _sc_degree(dst3, ones_rows, zeros_deg)
    h, dinv, c, s, y, zin = _tc1(x, degp, enc_W, encb,
                                 ne0_eW, e0b, ne0_dW, d0b)

    term, out = y, y
    for k in (1, 2, 3):
        aggp = _sc_aggregate(zin, src3, dst3, zeros_tile)
        term, out, zin = _tc_steps[k](term, out, aggp, dinv)
    aggp = _sc_aggregate(zin, src3, dst3, zeros_tile)
    h, c, s, y, zin = _tc_mid(term, out, aggp, dinv, c, s, h,
                              conv0_W, c0b, ne1_eW, e1b, ne1_dW, d1b)

    term, out = y, y
    for k in (1, 2, 3):
        aggp = _sc_aggregate(zin, src3, dst3, zeros_tile)
        term, out, zin = _tc_steps[k](term, out, aggp, dinv)
    aggp = _sc_aggregate(zin, src3, dst3, zeros_tile)
    (o,) = _tc_final(term, out, aggp, dinv, c, s, h, conv1_W, c1b,
                     dec_W, decb)
    return o[:N]


# speedup vs baseline: 1.0748x; 1.0748x over previous
import functools

import jax
import jax.numpy as jnp
from jax import lax
from jax.experimental import pallas as pl
from jax.experimental.pallas import tpu as pltpu
from jax.experimental.pallas import tpu_sc as plsc

N = 10000
E = 320000
IN_DIM = 128
H = 64
OUT_DIM = 40
TAU = 1.0
MAX_DEG = 4

NP = 10240
NW = 32
CHUNK = 128
NCH = 80
EPW = NCH * CHUNK
EPAD = NW * EPW
DEPTH = 5
RPT = NP // 16
DW = 16

_HI = lax.Precision.HIGHEST

_sc_mesh = plsc.VectorSubcoreMesh(core_axis_name="c", subcore_axis_name="s",
                                  num_cores=2, num_subcores=16)


@functools.partial(
    pl.kernel,
    out_type=jax.ShapeDtypeStruct((2, NP, H), jnp.float32),
    mesh=_sc_mesh,
    scratch_types=[
        pltpu.VMEM((NCH, CHUNK), jnp.int32),
        pltpu.VMEM((NCH, CHUNK), jnp.int32),
        [pltpu.VMEM((CHUNK, H), jnp.float32)] * DEPTH,
        pltpu.VMEM_SHARED((NP, H), jnp.float32),
        [pltpu.SemaphoreType.DMA] * DEPTH,
    ],
    compiler_params=pltpu.CompilerParams(use_tc_tiling_on_sc=False),
)
def _sc_aggregate(z_hbm, src_hbm, dst_hbm, zeros_hbm, out_hbm,
                  src_v, dst_v, bufs, agg, gsems):
    c = lax.axis_index("c")
    s = lax.axis_index("s")
    wid = c * 16 + s
    st0 = pltpu.async_copy(src_hbm.at[wid], src_v, gsems[0])
    st1 = pltpu.async_copy(dst_hbm.at[wid], dst_v, gsems[1])
    st2 = pltpu.async_copy(zeros_hbm, agg.at[pl.ds(s * RPT, RPT)], gsems[2])
    st0.wait()
    st1.wait()
    st2.wait()
    plsc.subcore_barrier()

    for b in range(DEPTH):
        pltpu.async_copy(z_hbm.at[src_v.at[b]], bufs[b], gsems[b])

    def body(g, carry):
        j0 = g * DEPTH
        for b in range(DEPTH):
            j = j0 + b
            pltpu.make_async_copy(z_hbm.at[src_v.at[j]], bufs[b],
                                  gsems[b]).wait()
            pltpu.sync_copy(bufs[b], agg.at[dst_v.at[j]], add=True)

            @pl.when(j + DEPTH < NCH)
            def _():
                pltpu.async_copy(z_hbm.at[src_v.at[j + DEPTH]], bufs[b],
                                 gsems[b])
        return carry

    lax.fori_loop(0, NCH // DEPTH, body, 0)
    plsc.subcore_barrier()
    pltpu.sync_copy(agg.at[pl.ds(s * RPT, RPT)],
                    out_hbm.at[c, pl.ds(s * RPT, RPT)])


@functools.partial(
    pl.kernel,
    out_type=jax.ShapeDtypeStruct((2, NP, DW), jnp.float32),
    mesh=_sc_mesh,
    scratch_types=[
        pltpu.VMEM((NCH, CHUNK), jnp.int32),
        pltpu.VMEM((CHUNK, DW), jnp.float32),
        pltpu.VMEM_SHARED((NP, DW), jnp.float32),
        [pltpu.SemaphoreType.DMA] * 2,
    ],
    compiler_params=pltpu.CompilerParams(use_tc_tiling_on_sc=False),
)
def _sc_degree(dst_hbm, ones_hbm, zeros_hbm, out_hbm, dst_v, ones_v, deg,
               sems):
    c = lax.axis_index("c")
    s = lax.axis_index("s")
    wid = c * 16 + s
    st0 = pltpu.async_copy(dst_hbm.at[wid], dst_v, sems[0])
    st1 = pltpu.async_copy(ones_hbm, ones_v, sems[1])
    pltpu.sync_copy(zeros_hbm, deg.at[pl.ds(s * RPT, RPT)])
    st0.wait()
    st1.wait()
    plsc.subcore_barrier()

    def body(j, carry):
        pltpu.sync_copy(ones_v, deg.at[dst_v.at[j]], add=True)
        return carry

    lax.fori_loop(0, NCH, body, 0)
    plsc.subcore_barrier()
    pltpu.sync_copy(deg.at[pl.ds(s * RPT, RPT)],
                    out_hbm.at[c, pl.ds(s * RPT, RPT)])


_GRID = 10
_BN = NP // _GRID


def _row_spec(d):
    return pl.BlockSpec((_BN, d), lambda i: (i, 0))


def _w_spec(a, b):
    return pl.BlockSpec((a, b), lambda i: (0, 0))


_AGG_SPEC = pl.BlockSpec((2, _BN, H), lambda i: (0, i, 0))
_DEG_SPEC = pl.BlockSpec((2, _BN, DW), lambda i: (0, i, 0))


def _rot_fwd(c, s, h):
    h0 = h[:, : H // 2]
    h1 = h[:, H // 2:]
    return jnp.concatenate([c * h0 + s * h1, c * h1 - s * h0], axis=1)


def _rot_bwd(c, s, z):
    z0 = z[:, : H // 2]
    z1 = z[:, H // 2:]
    return jnp.concatenate([c * z0 - s * z1, s * z0 + c * z1], axis=1)


def _theta_head(h, eW, eb, dW, db):
    t = jnp.dot(h, eW, preferred_element_type=jnp.float32, precision=_HI) + eb
    return jnp.dot(t, dW, preferred_element_type=jnp.float32, precision=_HI) + db


def _tc1_body(x_r, degp_r, encW_r, encb_r, eW_r, eb_r, dW_r, db_r,
              h_r, dinv_r, c_r, s_r, y_r, zin_r):
    deg2 = degp_r[0] + degp_r[1]
    dinv2 = jnp.where(deg2 > 0.0, 1.0 / jnp.sqrt(deg2), 0.0)
    dinv = dinv2[:, 0:1]
    h = jnp.dot(x_r[...], encW_r[...], preferred_element_type=jnp.float32,
                precision=_HI) + encb_r[...]
    theta = _theta_head(h, eW_r[...], eb_r[...], dW_r[...], db_r[...])
    c = jnp.cos(theta)
    s = jnp.sin(theta)
    y = _rot_fwd(c, s, h)
    h_r[...] = h
    dinv_r[...] = dinv
    c_r[...] = c
    s_r[...] = s
    y_r[...] = y
    zin_r[...] = dinv * y


_tc1 = pl.pallas_call(
    _tc1_body,
    grid=(_GRID,),
    in_specs=[_row_spec(IN_DIM), _DEG_SPEC, _w_spec(IN_DIM, H), _w_spec(1, H),
              _w_spec(H, H), _w_spec(1, H), _w_spec(H, 1), _w_spec(1, 1)],
    out_specs=[_row_spec(H), _row_spec(1), _row_spec(1), _row_spec(1),
               _row_spec(H), _row_spec(H)],
    out_shape=[jax.ShapeDtypeStruct((NP, H), jnp.float32),
               jax.ShapeDtypeStruct((NP, 1), jnp.float32),
               jax.ShapeDtypeStruct((NP, 1), jnp.float32),
               jax.ShapeDtypeStruct((NP, 1), jnp.float32),
               jax.ShapeDtypeStruct((NP, H), jnp.float32),
               jax.ShapeDtypeStruct((NP, H), jnp.float32)],
)


def _make_tc_step(k):
    def body(term_r, out_r, aggp_r, dinv_r, term2_r, out2_r, zin2_r):
        dinv = dinv_r[...]
        ahat = dinv * (aggp_r[0] + aggp_r[1])
        term2 = (-TAU / k) * (term_r[...] - ahat)
        term2_r[...] = term2
        out2_r[...] = out_r[...] + term2
        zin2_r[...] = dinv * term2

    return pl.pallas_call(
        body,
        grid=(_GRID,),
        in_specs=[_row_spec(H), _row_spec(H), _AGG_SPEC, _row_spec(1)],
        out_specs=[_row_spec(H), _row_spec(H), _row_spec(H)],
        out_shape=[jax.ShapeDtypeStruct((NP, H), jnp.float32)] * 3,
    )


_tc_steps = {k: _make_tc_step(k) for k in (1, 2, 3)}


def _finish_layer(term, out, aggp, dinv, c, s, h, convW, convb):
    ahat = dinv * (aggp[0] + aggp[1])
    z = out + (-TAU / MAX_DEG) * (term - ahat)
    w = _rot_bwd(c, s, z)
    hn = jnp.dot(w, convW, preferred_element_type=jnp.float32,
                 precision=_HI) + convb + h
    return jnp.maximum(hn, 0.0)


def _tc_mid_body(term_r, out_r, aggp_r, dinv_r, c_r, s_r, h_r,
                 convW_r, convb_r, eW_r, eb_r, dW_r, db_r,
                 hn_r, c1_r, s1_r, y1_r, zin1_r):
    dinv = dinv_r[...]
    hn = _finish_layer(term_r[...], out_r[...], aggp_r, dinv,
                       c_r[...], s_r[...], h_r[...], convW_r[...], convb_r[...])
    theta = _theta_head(hn, eW_r[...], eb_r[...], dW_r[...], db_r[...])
    c1 = jnp.cos(theta)
    s1 = jnp.sin(theta)
    y1 = _rot_fwd(c1, s1, hn)
    hn_r[...] = hn
    c1_r[...] = c1
    s1_r[...] = s1
    y1_r[...] = y1
    zin1_r[...] = dinv * y1


_tc_mid = pl.pallas_call(
    _tc_mid_body,
    grid=(_GRID,),
    in_specs=[_row_spec(H), _row_spec(H), _AGG_SPEC, _row_spec(1),
              _row_spec(1), _row_spec(1), _row_spec(H),
              _w_spec(H, H), _w_spec(1, H),
              _w_spec(H, H), _w_spec(1, H), _w_spec(H, 1), _w_spec(1, 1)],
    out_specs=[_row_spec(H), _row_spec(1), _row_spec(1), _row_spec(H),
               _row_spec(H)],
    out_shape=[jax.ShapeDtypeStruct((NP, H), jnp.float32),
               jax.ShapeDtypeStruct((NP, 1), jnp.float32),
               jax.ShapeDtypeStruct((NP, 1), jnp.float32),
               jax.ShapeDtypeStruct((NP, H), jnp.float32),
               jax.ShapeDtypeStruct((NP, H), jnp.float32)],
)


def _tc_final_body(term_r, out_r, aggp_r, dinv_r, c_r, s_r, h_r,
                   convW_r, convb_r, decW_r, decb_r, o_r):
    hn = _finish_layer(term_r[...], out_r[...], aggp_r, dinv_r[...],
                       c_r[...], s_r[...], h_r[...], convW_r[...], convb_r[...])
    o_r[...] = jnp.dot(hn, decW_r[...], preferred_element_type=jnp.float32,
                       precision=_HI) + decb_r[...]


_tc_final = pl.pallas_call(
    _tc_final_body,
    grid=(_GRID,),
    in_specs=[_row_spec(H), _row_spec(H), _AGG_SPEC, _row_spec(1),
              _row_spec(1), _row_spec(1), _row_spec(H),
              _w_spec(H, H), _w_spec(1, H),
              _w_spec(H, OUT_DIM), _w_spec(1, OUT_DIM)],
    out_specs=[_row_spec(OUT_DIM)],
    out_shape=[jax.ShapeDtypeStruct((NP, OUT_DIM), jnp.float32)],
)


def kernel(x, edge_index, enc_W, enc_b, ne0_eW, ne0_eb, ne0_dW, ne0_db,
           conv0_W, conv0_b, ne1_eW, ne1_eb, ne1_dW, ne1_db,
           conv1_W, conv1_b, dec_W, dec_b):
    f32 = jnp.float32
    pad_idx = (N + (jnp.arange(EPAD - E, dtype=jnp.int32) % (NP - N)))
    src3 = jnp.concatenate([edge_index[0], pad_idx]).reshape(NW, NCH, CHUNK)
    dst3 = jnp.concatenate([edge_index[1], pad_idx]).reshape(NW, NCH, CHUNK)
    x = jnp.pad(x, ((0, NP - N), (0, 0)))
    ones_tab = jnp.ones((NP, H), f32)
    zeros_tile = jnp.zeros((RPT, H), f32)
    ones_rows = jnp.ones((CHUNK, DW), f32)
    zeros_deg = jnp.zeros((RPT, DW), f32)

    encb = enc_b.reshape(1, H)
    e0b = ne0_eb.reshape(1, H)
    d0b = ne0_db.reshape(1, 1)
    c0b = conv0_b.reshape(1, H)
    e1b = ne1_eb.reshape(1, H)
    d1b = ne1_db.reshape(1, 1)
    c1b = conv1_b.reshape(1, H)
    decb = dec_b.reshape(1, OUT_DIM)

    degp = _sc_degree(dst3, ones_rows, zeros_deg)
    h, dinv, c, s, y, zin = _tc1(x, degp, enc_W, encb,
                                 ne0_eW, e0b, ne0_dW, d0b)

    term, out = y, y
    for k in (1, 2, 3):
        aggp = _sc_aggregate(zin, src3, dst3, zeros_tile)
        term, out, zin = _tc_steps[k](term, out, aggp, dinv)
    aggp = _sc_aggregate(zin, src3, dst3, zeros_tile)
    h, c, s, y, zin = _tc_mid(term, out, aggp, dinv, c, s, h,
                              conv0_W, c0b, ne1_eW, e1b, ne1_dW, d1b)

    term, out = y, y
    for k in (1, 2, 3):
        aggp = _sc_aggregate(zin, src3, dst3, zeros_tile)
        term, out, zin = _tc_steps[k](term, out, aggp, dinv)
    aggp = _sc_aggregate(zin, src3, dst3, zeros_tile)
    (o,) = _tc_final(term, out, aggp, dinv, c, s, h, conv1_W, c1b,
                     dec_W, decb)
    return o[:N]
